# baseline (speedup 1.0000x reference)
import functools

import jax
import jax.numpy as jnp
from jax import lax
from jax.experimental import pallas as pl
from jax.experimental.pallas import tpu as pltpu
from jax.experimental.pallas import tpu_sc as plsc

INPUT_DIM = 3
NUM_LEVELS = 16
LEVEL_DIM = 2
BASE_RES = 16
MAX_RES = 2048
LOG2_HASH = 19
HASH_SIZE = 2 ** LOG2_HASH
_b = (MAX_RES / BASE_RES) ** (1.0 / (NUM_LEVELS - 1))
_RESOLUTIONS = [float(int(BASE_RES * _b ** i)) for i in range(NUM_LEVELS)]
_P1 = -1640531535
_P2 = 805459861
_MASK = HASH_SIZE - 1
_NC = NUM_LEVELS * LEVEL_DIM
_TABF = NUM_LEVELS * HASH_SIZE * LEVEL_DIM

_GDN = lax.GatherDimensionNumbers(
    offset_dims=(), collapsed_slice_dims=(0,), start_index_map=(0,))


def _permute(v, idx):
    return lax.gather(v, idx[:, None], _GDN, slice_sizes=(1,),
                      mode=lax.GatherScatterMode.PROMISE_IN_BOUNDS)


def _pair_relayout_kernel(tabn_hbm, pairs_hbm, bin_, bout):
    NW = 32
    FW = _TABF // NW
    BK = 16384
    nblk = FW // BK

    cid = lax.axis_index("c")
    sid = lax.axis_index("s")
    wid = sid * 2 + cid
    iota = lax.broadcasted_iota(jnp.int32, (16,), 0)
    par = iota & 1
    half = iota >> 1

    def blk_body(k, _):
        off = wid * FW + k * BK
        pltpu.sync_copy(tabn_hbm.at[pl.ds(off, BK)], bin_)

        def v_body(j, _):
            sb = (j >> 4) * 256
            a0 = (j >> 1) & 7
            lidx = ((j & 1) * 8) + half
            av = bin_[pl.ds(sb + a0 * 16, 16)]
            bv = bin_[pl.ds(sb + 128 + a0 * 16, 16)]
            ag = _permute(av, lidx)
            bg = _permute(bv, lidx)
            bout[pl.ds(j * 16, 16)] = jnp.where(par == 1, bg, ag)
            return 0

        lax.fori_loop(jnp.int32(0), jnp.int32(BK // 16), v_body, 0)
        pltpu.sync_copy(bout, pairs_hbm.at[pl.ds(off, BK)])
        return 0

    lax.fori_loop(jnp.int32(0), jnp.int32(nblk), blk_body, 0)


def _hash_gather_kernel(B, C, xt_hbm, tab_hbm, out_hbm,
                        xv, i00, i01, i02, i03, i10, i11, i12, i13,
                        q00, q01, q02, q03, q10, q11, q12, q13,
                        r00, r01, r02, r03, r10, r11, r12, r13,
                        gsem0, gsem1, osem0, osem1):
    NW = 32
    PW = B // NW
    nchunk = PW // C
    PL = C * 8

    cid = lax.axis_index("c")
    sid = lax.axis_index("s")
    wid = sid * 2 + cid
    idx_b = ((i00, i01, i02, i03), (i10, i11, i12, i13))
    q_b = ((q00, q01, q02, q03), (q10, q11, q12, q13))
    rows_b = ((r00, r01, r02, r03), (r10, r11, r12, r13))
    gsem_b = (gsem0, gsem1)
    osem_b = (osem0, osem1)
    iota = lax.broadcasted_iota(jnp.int32, (16,), 0)
    perm2 = iota * 2
    permA = iota >> 1
    permB = permA + 8
    par = iota & 1

    def compute_idx(c, idx):
        base_p = wid * PW + c * C
        pltpu.sync_copy(xt_hbm.at[:, pl.ds(base_p, C)], xv)

        def grp_body(g, _):
            xa = xv[0, pl.ds(g * 16, 16)]
            xb = xv[1, pl.ds(g * 16, 16)]
            xc = xv[2, pl.ds(g * 16, 16)]
            goff = (g >> 3) * 1024 + (g & 7) * 32
            for l in range(NUM_LEVELS):
                r = _RESOLUTIONS[l]
                f0 = (xa * r).astype(jnp.int32)
                f1 = (xb * r).astype(jnp.int32)
                f2 = (xc * r).astype(jnp.int32)
                h = (f0 + f1 * _P1 + f2 * _P2) & _MASK
                parl = par + (l << (LOG2_HASH + 1))
                eA = (_permute(h, permA) << 1) + parl
                eB = (_permute(h, permB) << 1) + parl
                o = goff + (l & 3) * 256
                idx[l >> 2][pl.ds(o, 16)] = eA
                idx[l >> 2][pl.ds(o + 16, 16)] = eB
            return 0

        lax.fori_loop(jnp.int32(0), jnp.int32(C // 16), grp_body, 0)

    def start_gather(b):
        for j in range(4):
            pltpu.async_copy(tab_hbm.at[idx_b[b][j]], q_b[b][j], gsem_b[b])

    def wait_gather(b):
        for j in range(4):
            pltpu.make_async_copy(out_hbm.at[pl.ds(0, PL)],
                                  q_b[b][j], gsem_b[b]).wait()

    def deinterleave(b):
        def v_body(j, _):
            pb = j >> 6
            lq = (j >> 4) & 3
            d = (j >> 3) & 1
            prb = (j & 7) * 16
            base = pb * 1024 + lq * 256 + prb * 2
            lidx = (iota & 7) * 2 + d
            dst = pb * 1024 + (2 * lq + d) * 128 + prb
            for cb in range(4):
                qa = q_b[b][cb][pl.ds(base, 16)]
                qb = q_b[b][cb][pl.ds(base + 16, 16)]
                ag = _permute(qa, lidx)
                bg = _permute(qb, lidx)
                rows_b[b][cb][pl.ds(dst, 16)] = jnp.where(iota < 8, ag, bg)
            return 0

        lax.fori_loop(jnp.int32(0), jnp.int32(PL // 16), v_body, 0)

    def start_out(c, b):
        base_p = wid * PW + c * C
        for j in range(4):
            pltpu.async_copy(
                rows_b[b][j],
                out_hbm.at[pl.ds(j * (B * 8) + (base_p >> 7) * 1024, PL)],
                osem_b[b])

    def wait_out(b):
        for j in range(4):
            pltpu.make_async_copy(out_hbm.at[pl.ds(0, PL)],
                                  rows_b[b][j], osem_b[b]).wait()

    compute_idx(jnp.int32(0), idx_b[0])
    start_gather(0)
    compute_idx(jnp.int32(1), idx_b[1])
    wait_gather(0)
    start_gather(1)
    deinterleave(0)
    start_out(jnp.int32(0), 0)

    def pair_body(p, _):
        for b in range(2):
            c = 2 * p + b
            wait_out(b)
            compute_idx(c, idx_b[b])
            wait_gather(1 - b)
            start_gather(b)
            deinterleave(1 - b)
            start_out(c - 1, 1 - b)
        return 0

    lax.fori_loop(jnp.int32(1), jnp.int32(nchunk // 2), pair_body, 0)

    wait_gather(1)
    deinterleave(1)
    start_out(jnp.int32(nchunk - 1), 1)
    wait_out(0)
    wait_out(1)


def kernel(x, tables):
    B = x.shape[0]
    C = 512
    tabn = (tables.reshape(NUM_LEVELS, HASH_SIZE // 128, 128, LEVEL_DIM)
            .transpose(0, 1, 3, 2)
            .reshape(_TABF))
    PL = C * 8
    QL = C * 4

    mesh = plsc.VectorSubcoreMesh(core_axis_name="c", subcore_axis_name="s")
    pairs = pl.kernel(
        _pair_relayout_kernel,
        out_type=jax.ShapeDtypeStruct((_TABF,), jnp.float32),
        mesh=mesh,
        scratch_types=[
            pltpu.VMEM((16384,), jnp.float32),
            pltpu.VMEM((16384,), jnp.float32),
        ],
    )(tabn)

    out = pl.kernel(
        functools.partial(_hash_gather_kernel, B, C),
        out_type=jax.ShapeDtypeStruct((4 * B * 8,), jnp.float32),
        mesh=mesh,
        scratch_types=(
            [pltpu.VMEM((INPUT_DIM, C), jnp.float32)]
            + [pltpu.VMEM((PL,), jnp.int32) for _ in range(8)]
            + [pltpu.VMEM((PL,), jnp.float32) for _ in range(8)]
            + [pltpu.VMEM((PL,), jnp.float32) for _ in range(8)]
            + [pltpu.SemaphoreType.DMA for _ in range(4)]
        ),
    )(x.T, pairs)
    return (out.reshape(4, B // 128, 8, 128)
            .transpose(1, 3, 0, 2)
            .reshape(B, _NC))
